# initial kernel scaffold (unmeasured)
import jax
import jax.numpy as jnp
from jax import lax
from jax.experimental import pallas as pl
from jax.experimental.pallas import tpu as pltpu

N_DEV = 8
HPD = 8
DH = 128
SCALE = 0.08838834764831843


def _body(x_ref, k_ref, v_ref, wq_ref, wo_ref, out_ref,
          comm_ref, send_sems, recv_sems):
    my = lax.axis_index("i")
    left = jnp.mod(my - 1, N_DEV)
    right = jnp.mod(my + 1, N_DEV)

    barrier = pltpu.get_barrier_semaphore()
    for nbr in (left, right):
        pl.semaphore_signal(barrier, inc=1, device_id=(nbr,),
                            device_id_type=pl.DeviceIdType.MESH)
    pl.semaphore_wait(barrier, 2)

    comm_ref[0, 0:HPD] = wq_ref[...]
    comm_ref[0, HPD:2 * HPD] = wo_ref[...]

    x_bf = x_ref[...]
    sq = x_bf.shape[0]
    skv = k_ref.shape[1]
    row = lax.broadcasted_iota(jnp.int32, (sq, skv), 0)
    col = lax.broadcasted_iota(jnp.int32, (sq, skv), 1)
    mask = ((row // 64) % 4) == ((col // 64) % 4)

    def mm(a, b, dims):
        return lax.dot_general(a, b, dimension_numbers=(dims, ((), ())),
                               preferred_element_type=jnp.float32)

    acc = jnp.zeros(out_ref.shape, jnp.float32)

    for h in range(N_DEV):
        send_slot = h % 2
        recv_slot = (h + 1) % 2
        if h < N_DEV - 1:
            rdma = pltpu.make_async_remote_copy(
                src_ref=comm_ref.at[send_slot],
                dst_ref=comm_ref.at[recv_slot],
                send_sem=send_sems.at[send_slot],
                recv_sem=recv_sems.at[recv_slot],
                device_id=(right,),
                device_id_type=pl.DeviceIdType.MESH,
            )
            rdma.start()

        j = jnp.mod(my - h, N_DEV)

        def head_body(g_local, acc, slot=send_slot, j=j):
            g = j * HPD + g_local
            wq_g = comm_ref[slot, g_local]
            wo_g = comm_ref[slot, HPD + g_local]
            kh = k_ref[g]
            vh = v_ref[g]
            qh = mm(x_bf, wq_g, ((1,), (0,))).astype(jnp.bfloat16)
            scores = mm(qh, kh, ((1,), (1,))) * SCALE
            scores = jnp.where(mask, scores, -1e9)
            m = jnp.max(scores, axis=-1, keepdims=True)
            w = jnp.exp(scores - m)
            p = (w / jnp.sum(w, axis=-1, keepdims=True)).astype(jnp.bfloat16)
            ctx = mm(p, vh, ((1,), (0,))).astype(jnp.bfloat16)
            return acc + mm(ctx, wo_g, ((1,), (1,)))

        acc = lax.fori_loop(0, HPD, head_body, acc)

        if h < N_DEV - 1:
            rdma.wait()

    out_ref[...] = acc


def kernel(x, Wq, K_ext, V_ext, Wo):
    bf = jnp.bfloat16
    xb = x[0].astype(bf)
    wqh = Wq.astype(bf).reshape(Wq.shape[0], HPD, DH).transpose(1, 0, 2)
    kb = K_ext[0].astype(bf).transpose(1, 0, 2)
    vb = V_ext[0].astype(bf).transpose(1, 0, 2)
    woT = Wo.astype(bf).reshape(HPD, DH, Wo.shape[1]).transpose(0, 2, 1)

    sq = xb.shape[0]
    out = pl.pallas_call(
        _body,
        out_shape=jax.ShapeDtypeStruct((sq, Wo.shape[1]), jnp.float32),
        in_specs=[pl.BlockSpec(memory_space=pltpu.VMEM)] * 5,
        out_specs=pl.BlockSpec(memory_space=pltpu.VMEM),
        scratch_shapes=[
            pltpu.VMEM((2, 2 * HPD, Wq.shape[0], DH), bf),
            pltpu.SemaphoreType.DMA((2,)),
            pltpu.SemaphoreType.DMA((2,)),
        ],
        compiler_params=pltpu.CompilerParams(
            collective_id=0,
            vmem_limit_bytes=128 * 1024 * 1024,
        ),
    )(xb, kb, vb, wqh, woT)
    return out[None]


# baseline (device time: 434113 ns/iter reference)
import jax
import jax.numpy as jnp
from jax import lax
from jax.experimental import pallas as pl
from jax.experimental.pallas import tpu as pltpu

N_DEV = 8
HPD = 8
DH = 128
SCALE = 0.08838834764831843


def _body(x_ref, k_ref, v_ref, wq_ref, wo_ref, out_ref,
          comm_ref, kbuf, vbuf, send_sems, recv_sems, kcp_sems, vcp_sems):
    my = lax.axis_index("i")
    left = jnp.mod(my - 1, N_DEV)
    right = jnp.mod(my + 1, N_DEV)

    def kv_copies(h):
        j = jnp.mod(my - h, N_DEV)
        slot = h % 2
        kcp = pltpu.make_async_copy(
            k_ref.at[pl.ds(j * HPD, HPD)], kbuf.at[slot], kcp_sems.at[slot])
        vcp = pltpu.make_async_copy(
            v_ref.at[pl.ds(j * HPD, HPD)], vbuf.at[slot], vcp_sems.at[slot])
        return kcp, vcp

    for cp in kv_copies(0):
        cp.start()

    barrier = pltpu.get_barrier_semaphore()
    for nbr in (left, right):
        pl.semaphore_signal(barrier, inc=1, device_id=(nbr,),
                            device_id_type=pl.DeviceIdType.MESH)
    pl.semaphore_wait(barrier, 2)

    comm_ref[0, 0:HPD] = wq_ref[...]
    comm_ref[0, HPD:2 * HPD] = wo_ref[...]

    x_bf = x_ref[...]
    sq = x_bf.shape[0]
    skv = kbuf.shape[2]
    row = lax.broadcasted_iota(jnp.int32, (sq, skv), 0)
    col = lax.broadcasted_iota(jnp.int32, (sq, skv), 1)
    mask = ((row // 64) % 4) == ((col // 64) % 4)

    def mm(a, b, dims):
        return lax.dot_general(a, b, dimension_numbers=(dims, ((), ())),
                               preferred_element_type=jnp.float32)

    acc = jnp.zeros(out_ref.shape, jnp.float32)

    for h in range(N_DEV):
        send_slot = h % 2
        recv_slot = (h + 1) % 2
        if h < N_DEV - 1:
            rdma = pltpu.make_async_remote_copy(
                src_ref=comm_ref.at[send_slot],
                dst_ref=comm_ref.at[recv_slot],
                send_sem=send_sems.at[send_slot],
                recv_sem=recv_sems.at[recv_slot],
                device_id=(right,),
                device_id_type=pl.DeviceIdType.MESH,
            )
            rdma.start()
            for cp in kv_copies(h + 1):
                cp.start()

        for cp in kv_copies(h):
            cp.wait()

        def head_body(g_local, acc, slot=send_slot):
            wq_g = comm_ref[slot, g_local]
            wo_g = comm_ref[slot, HPD + g_local]
            kh = kbuf[slot, g_local]
            vh = vbuf[slot, g_local]
            qh = mm(x_bf, wq_g, ((1,), (0,))).astype(jnp.bfloat16)
            scores = mm(qh, kh, ((1,), (1,))) * SCALE
            scores = jnp.where(mask, scores, -1e9)
            m = jnp.max(scores, axis=-1, keepdims=True)
            w = jnp.exp(scores - m)
            p = (w / jnp.sum(w, axis=-1, keepdims=True)).astype(jnp.bfloat16)
            ctx = mm(p, vh, ((1,), (0,))).astype(jnp.bfloat16)
            return acc + mm(ctx, wo_g, ((1,), (1,)))

        acc = lax.fori_loop(0, HPD, head_body, acc)

        if h < N_DEV - 1:
            rdma.wait()

    out_ref[...] = acc


def kernel(x, Wq, K_ext, V_ext, Wo):
    bf = jnp.bfloat16
    xb = x[0].astype(bf)
    wqh = Wq.astype(bf).reshape(Wq.shape[0], HPD, DH).transpose(1, 0, 2)
    kb = K_ext[0].astype(bf).transpose(1, 0, 2)
    vb = V_ext[0].astype(bf).transpose(1, 0, 2)
    woT = Wo.astype(bf).reshape(HPD, DH, Wo.shape[1]).transpose(0, 2, 1)

    sq = xb.shape[0]
    skv = kb.shape[1]
    out = pl.pallas_call(
        _body,
        out_shape=jax.ShapeDtypeStruct((sq, Wo.shape[1]), jnp.float32),
        in_specs=[
            pl.BlockSpec(memory_space=pltpu.VMEM),
            pl.BlockSpec(memory_space=pltpu.MemorySpace.HBM),
            pl.BlockSpec(memory_space=pltpu.MemorySpace.HBM),
            pl.BlockSpec(memory_space=pltpu.VMEM),
            pl.BlockSpec(memory_space=pltpu.VMEM),
        ],
        out_specs=pl.BlockSpec(memory_space=pltpu.VMEM),
        scratch_shapes=[
            pltpu.VMEM((2, 2 * HPD, Wq.shape[0], DH), bf),
            pltpu.VMEM((2, HPD, skv, DH), bf),
            pltpu.VMEM((2, HPD, skv, DH), bf),
            pltpu.SemaphoreType.DMA((2,)),
            pltpu.SemaphoreType.DMA((2,)),
            pltpu.SemaphoreType.DMA((2,)),
            pltpu.SemaphoreType.DMA((2,)),
        ],
        compiler_params=pltpu.CompilerParams(
            collective_id=0,
            vmem_limit_bytes=63 * 1024 * 1024,
        ),
    )(xb, kb, vb, wqh, woT)
    return out[None]
